# baseline (speedup 1.0000x reference)
import functools

import jax
import jax.numpy as jnp
from jax import lax
from jax.experimental import pallas as pl
from jax.experimental.pallas import tpu as pltpu
from jax.experimental.pallas import tpu_sc as plsc

DIM = 64
MAXPOS = 200
LANE = 16
NCH = DIM // LANE
K = 128
UNROLL = 1
EPS = 1e-5


@functools.lru_cache(maxsize=None)
def _build(N):
    info = plsc.get_sparse_core_info()
    nc, ns = info.num_cores, info.num_subcores
    nw = nc * ns
    tok_w = N // nw
    nchunk = tok_w // K
    assert tok_w % K == 0 and nchunk % 2 == 0

    mesh = plsc.VectorSubcoreMesh(core_axis_name="c", subcore_axis_name="s")

    @functools.partial(
        pl.kernel,
        mesh=mesh,
        out_type=jax.ShapeDtypeStruct((N, DIM), jnp.float32),
        scratch_types=[
            pltpu.VMEM((nchunk, K), jnp.int32),
            pltpu.VMEM((2, K), jnp.int32),
            pltpu.VMEM((2, K), jnp.int32),
            pltpu.VMEM((K, 128), jnp.float32),
            pltpu.VMEM((K, 128), jnp.float32),
            pltpu.VMEM((K, DIM), jnp.float32),
            pltpu.VMEM((K, DIM), jnp.float32),
            pltpu.VMEM((MAXPOS, DIM), jnp.float32),
            pltpu.VMEM((DIM,), jnp.float32),
            pltpu.VMEM((DIM,), jnp.float32),
            pltpu.SemaphoreType.DMA,
            pltpu.SemaphoreType.DMA,
            pltpu.SemaphoreType.DMA,
            pltpu.SemaphoreType.DMA,
        ],
    )
    def emb(x_hbm, wt_hbm, pos_hbm, g_hbm, b_hbm, out_hbm,
            idx_v, pidx_v, half_v, rows0_v, rows1_v, o0_v, o1_v,
            pos_v, g_v, b_v, semg0, semg1, semw0, semw1):
        wid = lax.axis_index("s") * nc + lax.axis_index("c")
        base = wid * tok_w
        pltpu.sync_copy(x_hbm.at[pl.ds(wid * nchunk, nchunk)], idx_v)
        pltpu.sync_copy(pos_hbm, pos_v)
        pltpu.sync_copy(g_hbm, g_v)
        pltpu.sync_copy(b_hbm, b_v)
        gs = [g_v[pl.ds(c * LANE, LANE)] for c in range(NCH)]
        bs = [b_v[pl.ds(c * LANE, LANE)] for c in range(NCH)]
        lanes = lax.iota(jnp.int32, LANE)
        perms = [lanes ^ sh for sh in (8, 4, 2, 1)]
        rbufs = (rows0_v, rows1_v)
        obufs = (o0_v, o1_v)
        semg = (semg0, semg1)
        semw = (semw0, semw1)

        def lane_sum(v):
            for p in perms:
                v = v + v.at[p].get(mode="promise_in_bounds")
            return v

        def g_copy(par):
            return pltpu.make_async_copy(wt_hbm.at[pidx_v.at[par]],
                                         rbufs[par], semg[par])

        def w_copy(ci, par):
            off = pl.multiple_of(base + ci * K, K)
            return pltpu.make_async_copy(obufs[par], out_hbm.at[pl.ds(off, K)],
                                         semw[par])

        def fire(ci, par):
            for l in range(K // LANE):
                v = idx_v[ci, pl.ds(l * LANE, LANE)]
                pidx_v[par, pl.ds(l * LANE, LANE)] = v >> 1
                half_v[par, pl.ds(l * LANE, LANE)] = (v & 1) * DIM
            g_copy(par).start()

        def compute(ci, par):
            off = pl.multiple_of(base + ci * K, K)
            rows_v = rbufs[par]
            o_v = obufs[par]
            p_base = lax.rem(off, MAXPOS)

            def _tok(j, hb):
                pj = p_base + j
                pj = jnp.where(pj >= MAXPOS, pj - MAXPOS, pj)
                hs = [rows_v[j, pl.ds(hb + c * LANE, LANE)]
                      + pos_v[pj, pl.ds(c * LANE, LANE)] for c in range(NCH)]
                tot = (hs[0] + hs[1]) + (hs[2] + hs[3])
                sq = [h * h for h in hs]
                tsq = (sq[0] + sq[1]) + (sq[2] + sq[3])
                mean = lane_sum(tot) * (1.0 / DIM)
                ex2 = lane_sum(tsq) * (1.0 / DIM)
                v = ex2 - mean * mean + EPS
                bits = lax.bitcast_convert_type(v, jnp.int32)
                y = lax.bitcast_convert_type(
                    jnp.full((LANE,), 0x5F3759DF, jnp.int32) - (bits >> 1),
                    jnp.float32)
                vh = 0.5 * v
                y = y * (1.5 - vh * y * y)
                y = y * (1.5 - vh * y * y)
                for c in range(NCH):
                    o_v[j, pl.ds(c * LANE, LANE)] = \
                        (hs[c] - mean) * (y * gs[c]) + bs[c]

            @plsc.parallel_loop(0, K // LANE, 1, unroll=UNROLL)
            def tok16(t):
                hbvec = half_v[par, pl.ds(t * LANE, LANE)]
                for u in range(LANE):
                    _tok(t * LANE + u, hbvec[u])

        fire(0, 0)
        fire(1, 1)
        g_copy(0).wait()
        compute(0, 0)
        w_copy(0, 0).start()

        def pair(i, _):
            for h in range(2):
                ci = 1 + 2 * i + h
                par = 1 - h
                w_copy(ci - 1, h).wait()
                fire(ci + 1, h)
                g_copy(par).wait()
                compute(ci, par)
                w_copy(ci, par).start()
            return 0

        lax.fori_loop(0, (nchunk - 2) // 2, pair, 0)
        ci = nchunk - 1
        g_copy(1).wait()
        compute(ci, 1)
        w_copy(ci, 1).start()
        w_copy(ci - 1, 0).wait()
        w_copy(ci, 1).wait()

    return emb


def kernel(x, word_table, pos_table, gamma, beta):
    b, s = x.shape
    n = b * s
    v = word_table.shape[0]
    x2 = x.reshape(n // K, K)
    wt2 = word_table.reshape(v // 2, 2 * DIM)
    out = _build(n)(x2, wt2, pos_table, gamma, beta)
    return out.reshape(b, s, DIM)


# speedup vs baseline: 1.7007x; 1.7007x over previous
import functools

import jax
import jax.numpy as jnp
from jax import lax
from jax.experimental import pallas as pl
from jax.experimental.pallas import tpu as pltpu
from jax.experimental.pallas import tpu_sc as plsc

DIM = 64
MAXPOS = 200
LANE = 16
NCH = DIM // LANE
K = 128
UNROLL = 4
EPS = 1e-5


@functools.lru_cache(maxsize=None)
def _build(N):
    info = plsc.get_sparse_core_info()
    nc, ns = info.num_cores, info.num_subcores
    nw = nc * ns
    tok_w = N // nw
    nchunk = tok_w // K
    assert tok_w % K == 0 and nchunk % 2 == 0

    mesh = plsc.VectorSubcoreMesh(core_axis_name="c", subcore_axis_name="s")

    @functools.partial(
        pl.kernel,
        mesh=mesh,
        out_type=jax.ShapeDtypeStruct((N, DIM), jnp.float32),
        scratch_types=[
            pltpu.VMEM((nchunk, K), jnp.int32),
            pltpu.VMEM((K, 128), jnp.float32),
            pltpu.VMEM((K, 128), jnp.float32),
            pltpu.VMEM((K, DIM), jnp.float32),
            pltpu.VMEM((K, DIM), jnp.float32),
            pltpu.VMEM((MAXPOS, DIM), jnp.float32),
            pltpu.VMEM((DIM,), jnp.float32),
            pltpu.VMEM((DIM,), jnp.float32),
            pltpu.SemaphoreType.DMA,
            pltpu.SemaphoreType.DMA,
            pltpu.SemaphoreType.DMA,
            pltpu.SemaphoreType.DMA,
        ],
    )
    def emb(x_hbm, wt_hbm, pos_hbm, g_hbm, b_hbm, out_hbm,
            idx_v, rows0_v, rows1_v, o0_v, o1_v,
            pos_v, g_v, b_v, semg0, semg1, semw0, semw1):
        wid = lax.axis_index("s") * nc + lax.axis_index("c")
        base = wid * tok_w
        pltpu.sync_copy(x_hbm.at[pl.ds(wid * nchunk, nchunk)], idx_v)
        pltpu.sync_copy(pos_hbm, pos_v)
        pltpu.sync_copy(g_hbm, g_v)
        pltpu.sync_copy(b_hbm, b_v)
        gs = [g_v[pl.ds(c * LANE, LANE)] for c in range(NCH)]
        bs = [b_v[pl.ds(c * LANE, LANE)] for c in range(NCH)]
        lanes = lax.iota(jnp.int32, LANE)
        perms = [lanes ^ sh for sh in (8, 4, 2, 1)]
        rbufs = (rows0_v, rows1_v)
        obufs = (o0_v, o1_v)
        semg = (semg0, semg1)
        semw = (semw0, semw1)

        def lane_sum(v):
            for p in perms:
                v = v + v.at[p].get(mode="promise_in_bounds")
            return v

        def g_copy(ci, par):
            return pltpu.make_async_copy(wt_hbm.at[idx_v.at[ci]],
                                         rbufs[par], semg[par])

        def w_copy(ci, par):
            off = pl.multiple_of(base + ci * K, K)
            return pltpu.make_async_copy(obufs[par], out_hbm.at[pl.ds(off, K)],
                                         semw[par])

        def fire(ci, par):
            g_copy(ci, par).start()

        def compute(ci, par):
            off = pl.multiple_of(base + ci * K, K)
            rows_v = rbufs[par]
            o_v = obufs[par]
            p_base = lax.rem(off, MAXPOS)

            @plsc.parallel_loop(0, K, 1, unroll=UNROLL)
            def _tok(j):
                pj = p_base + j
                pj = jnp.where(pj >= MAXPOS, pj - MAXPOS, pj)
                hs = [rows_v[j, pl.ds(c * LANE, LANE)]
                      + pos_v[pj, pl.ds(c * LANE, LANE)] for c in range(NCH)]
                tot = (hs[0] + hs[1]) + (hs[2] + hs[3])
                sq = [h * h for h in hs]
                tsq = (sq[0] + sq[1]) + (sq[2] + sq[3])
                mean = lane_sum(tot) * (1.0 / DIM)
                ex2 = lane_sum(tsq) * (1.0 / DIM)
                v = ex2 - mean * mean + EPS
                bits = lax.bitcast_convert_type(v, jnp.int32)
                y = lax.bitcast_convert_type(
                    jnp.full((LANE,), 0x5F3759DF, jnp.int32) - (bits >> 1),
                    jnp.float32)
                vh = 0.5 * v
                y = y * (1.5 - vh * y * y)
                y = y * (1.5 - vh * y * y)
                for c in range(NCH):
                    o_v[j, pl.ds(c * LANE, LANE)] = \
                        (hs[c] - mean) * (y * gs[c]) + bs[c]

        fire(0, 0)
        fire(1, 1)
        g_copy(0, 0).wait()
        compute(0, 0)
        w_copy(0, 0).start()

        def pair(i, _):
            for h in range(2):
                ci = 1 + 2 * i + h
                par = 1 - h
                w_copy(ci - 1, h).wait()
                fire(ci + 1, h)
                g_copy(ci, par).wait()
                compute(ci, par)
                w_copy(ci, par).start()
            return 0

        lax.fori_loop(0, (nchunk - 2) // 2, pair, 0)
        ci = nchunk - 1
        g_copy(ci, 1).wait()
        compute(ci, 1)
        w_copy(ci, 1).start()
        w_copy(ci - 1, 0).wait()
        w_copy(ci, 1).wait()

    return emb


def kernel(x, word_table, pos_table, gamma, beta):
    b, s = x.shape
    n = b * s
    v = word_table.shape[0]
    x2 = x.reshape(n // K, K)
    wt2 = jnp.pad(word_table, ((0, 0), (0, DIM)))
    out = _build(n)(x2, wt2, pos_table, gamma, beta)
    return out.reshape(b, s, DIM)
